# baseline (speedup 1.0000x reference)
import dataclasses
import functools

import jax
import jax.numpy as jnp
from jax import lax
from jax.experimental import pallas as pl
from jax.experimental.pallas import tpu as pltpu
from jax.experimental.pallas import tpu_sc as plsc

N = 10000
E = 320000
NF = 128
NCLS = 40

NUM_SC = 2
NUM_SUB = 16
NW = NUM_SC * NUM_SUB

NPAD = 10112
CPAD = 48
BLK = 128
EPW = E // NW
EBF = EPW // BLK
TAIL = EPW - EBF * BLK
RPS = NPAD // NUM_SUB

_mesh = plsc.VectorSubcoreMesh(
    core_axis_name="c", subcore_axis_name="s",
    num_cores=NUM_SC, num_subcores=NUM_SUB,
)


def _hist_body(ei_hbm, out_hbm, idx_v, hist_v):
    wid = lax.axis_index("s") * NUM_SC + lax.axis_index("c")
    pltpu.sync_copy(ei_hbm.at[1, pl.ds(wid * EPW, EPW)], idx_v)
    zeros16 = jnp.zeros((16,), jnp.float32)

    @pl.loop(0, NPAD // 16)
    def _(i):
        hist_v[pl.ds(i * 16, 16)] = zeros16

    ones16 = jnp.ones((16,), jnp.float32)

    @pl.loop(0, EPW // 16)
    def _(i):
        idx = idx_v[pl.ds(i * 16, 16)]
        plsc.addupdate_scatter(hist_v, [idx], ones16)

    pltpu.sync_copy(hist_v, out_hbm.at[wid])


_sc_params = pltpu.CompilerParams(
    needs_layout_passes=False, use_tc_tiling_on_sc=False
)

_hist_call = functools.partial(
    pl.kernel,
    out_type=jax.ShapeDtypeStruct((NW, NPAD), jnp.float32),
    mesh=_mesh,
    compiler_params=_sc_params,
    scratch_types=[
        pltpu.VMEM((EPW,), jnp.int32),
        pltpu.VMEM((NPAD,), jnp.float32),
    ],
)(_hist_body)


def _h_body(x_ref, w_ref, h_ref):
    h = jnp.dot(x_ref[...], w_ref[...], preferred_element_type=jnp.float32)
    h_ref[...] = jnp.pad(h, ((0, NPAD - N), (0, CPAD - NCLS)))


def _h_call(x, w):
    return pl.pallas_call(
        _h_body,
        out_shape=jax.ShapeDtypeStruct((NPAD, CPAD), jnp.float32),
    )(x, w)


_GB = 128


def _g_body(h_ref, hist_ref, g_ref, dinv_ref):
    ones = jnp.ones((NW, 1), jnp.float32)
    deg = lax.dot_general(
        hist_ref[...], ones,
        dimension_numbers=(((0,), (0,)), ((), ())),
        preferred_element_type=jnp.float32,
    ) + 1.0
    dinv = lax.rsqrt(deg)
    dinv_ref[...] = dinv
    g_ref[...] = h_ref[...] * dinv


def _g_call(h, hist):
    return pl.pallas_call(
        _g_body,
        out_shape=[
            jax.ShapeDtypeStruct((NPAD, CPAD), jnp.float32),
            jax.ShapeDtypeStruct((NPAD, 1), jnp.float32),
        ],
    )(h, hist)


def _blk(v_ref, j):
    return v_ref.at[pl.ds(pl.multiple_of(j * BLK, BLK), BLK)]


NBUF = 4


def _main_body(g_hbm, ei_hbm, zero_hbm, out_hbm,
               src_v, dst_v, rows0, rows1, rows2, rows3, g_sh, acc_sh,
               gsem0, gsem1, gsem2, gsem3, ssem0, ssem1, ssem2, ssem3):
    rows = [rows0, rows1, rows2, rows3]
    gsem = [gsem0, gsem1, gsem2, gsem3]
    ssem = [ssem0, ssem1, ssem2, ssem3]
    c = lax.axis_index("c")
    s = lax.axis_index("s")
    wid = s * NUM_SC + c
    base = wid * EPW
    pltpu.sync_copy(zero_hbm.at[pl.ds(s * RPS, RPS)],
                    acc_sh.at[pl.ds(s * RPS, RPS)])
    pltpu.sync_copy(g_hbm.at[pl.ds(s * RPS, RPS)],
                    g_sh.at[pl.ds(s * RPS, RPS)])
    pltpu.sync_copy(ei_hbm.at[0, pl.ds(base, EPW)], src_v)
    pltpu.sync_copy(ei_hbm.at[1, pl.ds(base, EPW)], dst_v)
    plsc.subcore_barrier()

    for k in range(NBUF):
        pltpu.async_copy(g_sh.at[_blk(src_v, k)], rows[k], gsem[k])

    @pl.loop(0, 80, step=NBUF)
    def _(j):
        for k in range(NBUF):
            jk = j + k

            @pl.when(jk < EBF)
            def _(k=k, jk=jk):
                pltpu.make_async_copy(
                    g_sh.at[_blk(src_v, 0)], rows[k], gsem[k]).wait()
                pltpu.async_copy(rows[k], acc_sh.at[_blk(dst_v, jk)],
                                 ssem[k], add=True)

        for k in range(NBUF):
            jk = j + k

            @pl.when(jk + NBUF < EBF)
            def _(k=k, jk=jk):
                pltpu.make_async_copy(
                    rows[k], acc_sh.at[_blk(dst_v, 0)], ssem[k]).wait()
                pltpu.async_copy(g_sh.at[_blk(src_v, jk + NBUF)],
                                 rows[k], gsem[k])

    for k in range(NBUF):
        pltpu.make_async_copy(
            rows[k], acc_sh.at[_blk(dst_v, 0)], ssem[k]).wait()

    tail = pl.ds(EBF * BLK, TAIL)
    pltpu.sync_copy(g_sh.at[src_v.at[tail]], rows0.at[pl.ds(0, TAIL)])
    pltpu.sync_copy(rows0.at[pl.ds(0, TAIL)], acc_sh.at[dst_v.at[tail]],
                    add=True)

    plsc.subcore_barrier()
    pltpu.sync_copy(acc_sh.at[pl.ds(s * RPS, RPS)],
                    out_hbm.at[c, pl.ds(s * RPS, RPS)])


_main_call = functools.partial(
    pl.kernel,
    out_type=jax.ShapeDtypeStruct((NUM_SC, NPAD, CPAD), jnp.float32),
    mesh=_mesh,
    compiler_params=_sc_params,
    scratch_types=(
        [pltpu.VMEM((EPW,), jnp.int32)] * 2
        + [pltpu.VMEM((BLK, CPAD), jnp.float32)] * NBUF
        + [pltpu.VMEM_SHARED((NPAD, CPAD), jnp.float32)] * 2
        + [pltpu.SemaphoreType.DMA] * (2 * NBUF)
    ),
)(_main_body)


_FB = 1000


def _final_body(acc_ref, g_ref, dinv_ref, b_ref, o_ref):
    total = acc_ref[0] + acc_ref[1] + g_ref[...]
    z = total[:, :NCLS] * dinv_ref[...] + b_ref[...]
    m = jnp.max(z, axis=1, keepdims=True)
    lse = jnp.log(jnp.sum(jnp.exp(z - m), axis=1, keepdims=True))
    o_ref[...] = z - m - lse


def _final_call(acc, g, dinv, b):
    return pl.pallas_call(
        _final_body,
        grid=(N // _FB,),
        in_specs=[
            pl.BlockSpec((NUM_SC, _FB, CPAD), lambda i: (0, i, 0)),
            pl.BlockSpec((_FB, CPAD), lambda i: (i, 0)),
            pl.BlockSpec((_FB, 1), lambda i: (i, 0)),
            pl.BlockSpec((1, NCLS), lambda i: (0, 0)),
        ],
        out_specs=pl.BlockSpec((_FB, NCLS), lambda i: (i, 0)),
        out_shape=jax.ShapeDtypeStruct((N, NCLS), jnp.float32),
    )(acc, g, dinv, b)


@jax.jit
def kernel(x, edge_index, W, b):
    ei = edge_index.astype(jnp.int32)
    zeros = jnp.zeros((NPAD, CPAD), jnp.float32)

    hist = _hist_call(ei)
    h = _h_call(x.astype(jnp.float32), W.astype(jnp.float32))
    g, dinv = _g_call(h, hist)
    acc = _main_call(g, ei, zeros)
    return _final_call(acc, g, dinv, b.reshape(1, NCLS))


# speedup vs baseline: 1.0904x; 1.0904x over previous
import dataclasses
import functools

import jax
import jax.numpy as jnp
from jax import lax
from jax.experimental import pallas as pl
from jax.experimental.pallas import tpu as pltpu
from jax.experimental.pallas import tpu_sc as plsc

N = 10000
E = 320000
NF = 128
NCLS = 40

NUM_SC = 2
NUM_SUB = 16
NW = NUM_SC * NUM_SUB

NPAD = 10112
CPAD = 48
BLK = 128
EPW = E // NW
EBF = EPW // BLK
TAIL = EPW - EBF * BLK
RPS = NPAD // NUM_SUB

_mesh = plsc.VectorSubcoreMesh(
    core_axis_name="c", subcore_axis_name="s",
    num_cores=NUM_SC, num_subcores=NUM_SUB,
)


def _hist_body(ei_hbm, out_hbm, idx_v, hist_v):
    wid = lax.axis_index("s") * NUM_SC + lax.axis_index("c")
    pltpu.sync_copy(ei_hbm.at[1, pl.ds(wid * EPW, EPW)], idx_v)
    zeros16 = jnp.zeros((16,), jnp.float32)

    @pl.loop(0, NPAD // 16, step=8)
    def _(i):
        for u in range(8):
            hist_v[pl.ds(pl.multiple_of((i + u) * 16, 16), 16)] = zeros16

    ones16 = jnp.ones((16,), jnp.float32)

    @pl.loop(0, EPW // 16, step=5)
    def _(i):
        for u in range(5):
            idx = idx_v[pl.ds(pl.multiple_of((i + u) * 16, 16), 16)]
            plsc.addupdate_scatter(hist_v, [idx], ones16)

    pltpu.sync_copy(hist_v, out_hbm.at[wid])


_sc_params = pltpu.CompilerParams(
    needs_layout_passes=False, use_tc_tiling_on_sc=False
)

_hist_call = functools.partial(
    pl.kernel,
    out_type=jax.ShapeDtypeStruct((NW, NPAD), jnp.float32),
    mesh=_mesh,
    compiler_params=_sc_params,
    scratch_types=[
        pltpu.VMEM((EPW,), jnp.int32),
        pltpu.VMEM((NPAD,), jnp.float32),
    ],
)(_hist_body)


def _h_body(x_ref, w_ref, h_ref):
    h = jnp.dot(x_ref[...], w_ref[...], preferred_element_type=jnp.float32)
    h_ref[...] = jnp.pad(h, ((0, NPAD - N), (0, CPAD - NCLS)))


def _h_call(x, w):
    return pl.pallas_call(
        _h_body,
        out_shape=jax.ShapeDtypeStruct((NPAD, CPAD), jnp.float32),
    )(x, w)


_GB = 128


def _g_body(h_ref, hist_ref, g_ref, dinv_ref):
    ones = jnp.ones((NW, 1), jnp.float32)
    deg = lax.dot_general(
        hist_ref[...], ones,
        dimension_numbers=(((0,), (0,)), ((), ())),
        preferred_element_type=jnp.float32,
    ) + 1.0
    dinv = lax.rsqrt(deg)
    dinv_ref[...] = dinv
    g_ref[...] = h_ref[...] * dinv


def _g_call(h, hist):
    return pl.pallas_call(
        _g_body,
        out_shape=[
            jax.ShapeDtypeStruct((NPAD, CPAD), jnp.float32),
            jax.ShapeDtypeStruct((NPAD, 1), jnp.float32),
        ],
    )(h, hist)


def _blk(v_ref, j):
    return v_ref.at[pl.ds(pl.multiple_of(j * BLK, BLK), BLK)]


def _main_body(g_hbm, ei_hbm, zero_hbm, out_hbm,
               src_v, dst_v, rows0, rows1, g_sh, acc_sh, sem0, sem1):
    c = lax.axis_index("c")
    s = lax.axis_index("s")
    wid = s * NUM_SC + c
    base = wid * EPW
    pltpu.sync_copy(zero_hbm.at[pl.ds(s * RPS, RPS)],
                    acc_sh.at[pl.ds(s * RPS, RPS)])
    pltpu.sync_copy(g_hbm.at[pl.ds(s * RPS, RPS)],
                    g_sh.at[pl.ds(s * RPS, RPS)])
    pltpu.sync_copy(ei_hbm.at[0, pl.ds(base, EPW)], src_v)
    pltpu.sync_copy(ei_hbm.at[1, pl.ds(base, EPW)], dst_v)
    plsc.subcore_barrier()

    pltpu.async_copy(g_sh.at[_blk(src_v, 0)], rows0, sem0)
    pltpu.async_copy(g_sh.at[_blk(src_v, 1)], rows1, sem1)

    @pl.loop(0, EBF, step=2)
    def _(j):
        pltpu.make_async_copy(g_sh.at[_blk(src_v, 0)], rows0, sem0).wait()
        pltpu.sync_copy(rows0, acc_sh.at[_blk(dst_v, j)], add=True)

        @pl.when(j + 2 < EBF)
        def _():
            pltpu.async_copy(g_sh.at[_blk(src_v, j + 2)], rows0, sem0)

        pltpu.make_async_copy(g_sh.at[_blk(src_v, 1)], rows1, sem1).wait()
        pltpu.sync_copy(rows1, acc_sh.at[_blk(dst_v, j + 1)], add=True)

        @pl.when(j + 3 < EBF)
        def _():
            pltpu.async_copy(g_sh.at[_blk(src_v, j + 3)], rows1, sem1)

    tail = pl.ds(EBF * BLK, TAIL)
    pltpu.sync_copy(g_sh.at[src_v.at[tail]], rows0.at[pl.ds(0, TAIL)])
    pltpu.sync_copy(rows0.at[pl.ds(0, TAIL)], acc_sh.at[dst_v.at[tail]],
                    add=True)

    plsc.subcore_barrier()
    pltpu.sync_copy(acc_sh.at[pl.ds(s * RPS, RPS)],
                    out_hbm.at[c, pl.ds(s * RPS, RPS)])


_main_call = functools.partial(
    pl.kernel,
    out_type=jax.ShapeDtypeStruct((NUM_SC, NPAD, CPAD), jnp.float32),
    mesh=_mesh,
    compiler_params=_sc_params,
    scratch_types=[
        pltpu.VMEM((EPW,), jnp.int32),
        pltpu.VMEM((EPW,), jnp.int32),
        pltpu.VMEM((BLK, CPAD), jnp.float32),
        pltpu.VMEM((BLK, CPAD), jnp.float32),
        pltpu.VMEM_SHARED((NPAD, CPAD), jnp.float32),
        pltpu.VMEM_SHARED((NPAD, CPAD), jnp.float32),
        pltpu.SemaphoreType.DMA,
        pltpu.SemaphoreType.DMA,
    ],
)(_main_body)


_FB = 1000


def _final_body(acc_ref, g_ref, dinv_ref, b_ref, o_ref):
    total = acc_ref[0] + acc_ref[1] + g_ref[...]
    z = total[:, :NCLS] * dinv_ref[...] + b_ref[...]
    m = jnp.max(z, axis=1, keepdims=True)
    lse = jnp.log(jnp.sum(jnp.exp(z - m), axis=1, keepdims=True))
    o_ref[...] = z - m - lse


def _final_call(acc, g, dinv, b):
    return pl.pallas_call(
        _final_body,
        grid=(N // _FB,),
        in_specs=[
            pl.BlockSpec((NUM_SC, _FB, CPAD), lambda i: (0, i, 0)),
            pl.BlockSpec((_FB, CPAD), lambda i: (i, 0)),
            pl.BlockSpec((_FB, 1), lambda i: (i, 0)),
            pl.BlockSpec((1, NCLS), lambda i: (0, 0)),
        ],
        out_specs=pl.BlockSpec((_FB, NCLS), lambda i: (i, 0)),
        out_shape=jax.ShapeDtypeStruct((N, NCLS), jnp.float32),
    )(acc, g, dinv, b)


@jax.jit
def kernel(x, edge_index, W, b):
    ei = edge_index.astype(jnp.int32)
    zeros = jnp.zeros((NPAD, CPAD), jnp.float32)

    hist = _hist_call(ei)
    h = _h_call(x.astype(jnp.float32), W.astype(jnp.float32))
    g, dinv = _g_call(h, hist)
    acc = _main_call(g, ei, zeros)
    return _final_call(acc, g, dinv, b.reshape(1, NCLS))


# speedup vs baseline: 1.1058x; 1.0140x over previous
import dataclasses
import functools

import jax
import jax.numpy as jnp
from jax import lax
from jax.experimental import pallas as pl
from jax.experimental.pallas import tpu as pltpu
from jax.experimental.pallas import tpu_sc as plsc

N = 10000
E = 320000
NF = 128
NCLS = 40

NUM_SC = 2
NUM_SUB = 16
NW = NUM_SC * NUM_SUB

NPAD = 10112
CPAD = 48
BLK = 128
EPW = E // NW
EBF = EPW // BLK
TAIL = EPW - EBF * BLK
RPS = NPAD // NUM_SUB

_mesh = plsc.VectorSubcoreMesh(
    core_axis_name="c", subcore_axis_name="s",
    num_cores=NUM_SC, num_subcores=NUM_SUB,
)


def _hist_body(ei_hbm, out_hbm, idx_v, hist_v, sem):
    wid = lax.axis_index("s") * NUM_SC + lax.axis_index("c")
    cp = pltpu.async_copy(ei_hbm.at[1, pl.ds(wid * EPW, EPW)], idx_v, sem)
    zeros16 = jnp.zeros((16,), jnp.float32)

    @pl.loop(0, NPAD // 16, step=8)
    def _(i):
        for u in range(8):
            hist_v[pl.ds(pl.multiple_of((i + u) * 16, 16), 16)] = zeros16

    ones16 = jnp.ones((16,), jnp.float32)
    cp.wait()

    @pl.loop(0, EPW // 16, step=5)
    def _(i):
        for u in range(5):
            idx = idx_v[pl.ds(pl.multiple_of((i + u) * 16, 16), 16)]
            plsc.addupdate_scatter(hist_v, [idx], ones16)

    pltpu.sync_copy(hist_v, out_hbm.at[wid])


_sc_params = pltpu.CompilerParams(
    needs_layout_passes=False, use_tc_tiling_on_sc=False
)

_hist_call = functools.partial(
    pl.kernel,
    out_type=jax.ShapeDtypeStruct((NW, NPAD), jnp.float32),
    mesh=_mesh,
    compiler_params=_sc_params,
    scratch_types=[
        pltpu.VMEM((EPW,), jnp.int32),
        pltpu.VMEM((NPAD,), jnp.float32),
        pltpu.SemaphoreType.DMA,
    ],
)(_hist_body)


def _h_body(x_ref, w_ref, h_ref):
    h = jnp.dot(x_ref[...], w_ref[...], preferred_element_type=jnp.float32)
    h_ref[...] = jnp.pad(h, ((0, NPAD - N), (0, CPAD - NCLS)))


def _h_call(x, w):
    return pl.pallas_call(
        _h_body,
        out_shape=jax.ShapeDtypeStruct((NPAD, CPAD), jnp.float32),
    )(x, w)


_GB = 128


def _g_body(h_ref, hist_ref, g_ref, dinv_ref):
    ones = jnp.ones((NW, 1), jnp.float32)
    deg = lax.dot_general(
        hist_ref[...], ones,
        dimension_numbers=(((0,), (0,)), ((), ())),
        preferred_element_type=jnp.float32,
    ) + 1.0
    dinv = lax.rsqrt(deg)
    dinv_ref[...] = dinv
    g_ref[...] = h_ref[...] * dinv


def _g_call(h, hist):
    return pl.pallas_call(
        _g_body,
        out_shape=[
            jax.ShapeDtypeStruct((NPAD, CPAD), jnp.float32),
            jax.ShapeDtypeStruct((NPAD, 1), jnp.float32),
        ],
    )(h, hist)


def _blk(v_ref, j):
    return v_ref.at[pl.ds(pl.multiple_of(j * BLK, BLK), BLK)]


def _main_body(g_hbm, ei_hbm, zero_hbm, out_hbm,
               src_v, dst_v, rows0, rows1, g_sh, acc_sh, sem0, sem1):
    c = lax.axis_index("c")
    s = lax.axis_index("s")
    wid = s * NUM_SC + c
    base = wid * EPW
    cp1 = pltpu.async_copy(zero_hbm.at[pl.ds(s * RPS, RPS)],
                           acc_sh.at[pl.ds(s * RPS, RPS)], sem0)
    cp2 = pltpu.async_copy(g_hbm.at[pl.ds(s * RPS, RPS)],
                           g_sh.at[pl.ds(s * RPS, RPS)], sem1)
    cp3 = pltpu.async_copy(ei_hbm.at[0, pl.ds(base, EPW)], src_v, sem0)
    cp4 = pltpu.async_copy(ei_hbm.at[1, pl.ds(base, EPW)], dst_v, sem1)
    cp1.wait()
    cp2.wait()
    cp3.wait()
    cp4.wait()
    plsc.subcore_barrier()

    pltpu.async_copy(g_sh.at[_blk(src_v, 0)], rows0, sem0)
    pltpu.async_copy(g_sh.at[_blk(src_v, 1)], rows1, sem1)

    @pl.loop(0, EBF, step=2)
    def _(j):
        pltpu.make_async_copy(g_sh.at[_blk(src_v, 0)], rows0, sem0).wait()
        pltpu.sync_copy(rows0, acc_sh.at[_blk(dst_v, j)], add=True)

        @pl.when(j + 2 < EBF)
        def _():
            pltpu.async_copy(g_sh.at[_blk(src_v, j + 2)], rows0, sem0)

        pltpu.make_async_copy(g_sh.at[_blk(src_v, 1)], rows1, sem1).wait()
        pltpu.sync_copy(rows1, acc_sh.at[_blk(dst_v, j + 1)], add=True)

        @pl.when(j + 3 < EBF)
        def _():
            pltpu.async_copy(g_sh.at[_blk(src_v, j + 3)], rows1, sem1)

    tail = pl.ds(EBF * BLK, TAIL)
    pltpu.sync_copy(g_sh.at[src_v.at[tail]], rows0.at[pl.ds(0, TAIL)])
    pltpu.sync_copy(rows0.at[pl.ds(0, TAIL)], acc_sh.at[dst_v.at[tail]],
                    add=True)

    plsc.subcore_barrier()
    pltpu.sync_copy(acc_sh.at[pl.ds(s * RPS, RPS)],
                    out_hbm.at[c, pl.ds(s * RPS, RPS)])


_main_call = functools.partial(
    pl.kernel,
    out_type=jax.ShapeDtypeStruct((NUM_SC, NPAD, CPAD), jnp.float32),
    mesh=_mesh,
    compiler_params=_sc_params,
    scratch_types=[
        pltpu.VMEM((EPW,), jnp.int32),
        pltpu.VMEM((EPW,), jnp.int32),
        pltpu.VMEM((BLK, CPAD), jnp.float32),
        pltpu.VMEM((BLK, CPAD), jnp.float32),
        pltpu.VMEM_SHARED((NPAD, CPAD), jnp.float32),
        pltpu.VMEM_SHARED((NPAD, CPAD), jnp.float32),
        pltpu.SemaphoreType.DMA,
        pltpu.SemaphoreType.DMA,
    ],
)(_main_body)


_FB = 1000


def _final_body(acc_ref, g_ref, dinv_ref, b_ref, o_ref):
    total = acc_ref[0] + acc_ref[1] + g_ref[...]
    z = total[:, :NCLS] * dinv_ref[...] + b_ref[...]
    m = jnp.max(z, axis=1, keepdims=True)
    lse = jnp.log(jnp.sum(jnp.exp(z - m), axis=1, keepdims=True))
    o_ref[...] = z - m - lse


def _final_call(acc, g, dinv, b):
    return pl.pallas_call(
        _final_body,
        grid=(N // _FB,),
        in_specs=[
            pl.BlockSpec((NUM_SC, _FB, CPAD), lambda i: (0, i, 0)),
            pl.BlockSpec((_FB, CPAD), lambda i: (i, 0)),
            pl.BlockSpec((_FB, 1), lambda i: (i, 0)),
            pl.BlockSpec((1, NCLS), lambda i: (0, 0)),
        ],
        out_specs=pl.BlockSpec((_FB, NCLS), lambda i: (i, 0)),
        out_shape=jax.ShapeDtypeStruct((N, NCLS), jnp.float32),
    )(acc, g, dinv, b)


@jax.jit
def kernel(x, edge_index, W, b):
    ei = edge_index.astype(jnp.int32)
    zeros = jnp.zeros((NPAD, CPAD), jnp.float32)

    hist = _hist_call(ei)
    h = _h_call(x.astype(jnp.float32), W.astype(jnp.float32))
    g, dinv = _g_call(h, hist)
    acc = _main_call(g, ei, zeros)
    return _final_call(acc, g, dinv, b.reshape(1, NCLS))
